# Initial kernel scaffold; baseline (speedup 1.0000x reference)
import jax
import jax.numpy as jnp
from jax.experimental import pallas as pl


def kernel(x, edge_index, batch, bnf_g, bnf_b, W_feat, b_feat, bns_g, bns_b, Ws, bs, bnfc_g, bnfc_b, W_fc, b_fc, bnh_g, bnh_b, W_cls, b_cls):
    raise NotImplementedError("write your pallas kernel here")


# speedup vs baseline: 10.2661x; 10.2661x over previous
import functools

import jax
import jax.numpy as jnp
from jax import lax
from jax.experimental import pallas as pl
from jax.experimental.pallas import tpu as pltpu
from jax.experimental.pallas import tpu_sc as plsc

F = 128
NCORES = 2
NSUB = 16
LANES = 128
RPT_ALIGN = 8

_HIGH = jax.lax.Precision.HIGHEST


def _mm(a, b):
    return jnp.dot(a, b, precision=_HIGH, preferred_element_type=jnp.float32)


def _bn_rows(x, g, b):
    m = jnp.mean(x, axis=0, keepdims=True)
    v = jnp.mean((x - m) ** 2, axis=0, keepdims=True)
    return (x - m) / jnp.sqrt(v + 1e-5) * g + b


def _sc_mesh():
    return plsc.VectorSubcoreMesh(core_axis_name="c", subcore_axis_name="s")


def _make_sc_deg(np_, ch, rpt):
    @functools.partial(
        pl.kernel,
        out_type=jax.ShapeDtypeStruct((NCORES * np_,), jnp.float32),
        mesh=_sc_mesh(),
        scratch_types=[
            pltpu.VMEM((ch, LANES), jnp.int32),
            pltpu.VMEM((LANES,), jnp.float32),
            pltpu.VMEM((rpt,), jnp.float32),
            pltpu.MemorySpace.VMEM_SHARED((np_,), jnp.float32),
            pltpu.SemaphoreType.DMA,
        ],
    )
    def sc_deg(row_hbm, ones_hbm, out_hbm, row_v, ones_v, stage_v, deg_sh, sem):
        c = lax.axis_index("c")
        s = lax.axis_index("s")
        slab = s * NCORES + c
        pltpu.sync_copy(row_hbm.at[slab], row_v)
        pltpu.sync_copy(ones_hbm.at[pl.ds(0, LANES)], ones_v)
        pltpu.sync_copy(ones_hbm, stage_v)
        pltpu.sync_copy(stage_v, deg_sh.at[pl.ds(s * rpt, rpt)])
        plsc.subcore_barrier()

        def body(j, carry):
            pltpu.sync_copy(ones_v, deg_sh.at[row_v.at[j]], add=True)
            return carry

        lax.fori_loop(0, ch, body, 0)
        plsc.subcore_barrier()
        pltpu.sync_copy(deg_sh.at[pl.ds(s * rpt, rpt)], stage_v)
        pltpu.sync_copy(stage_v, out_hbm.at[pl.ds(c * np_ + s * rpt, rpt)])

    return sc_deg


def _make_sc_prop(np_, ch, rpt):
    @functools.partial(
        pl.kernel,
        out_type=jax.ShapeDtypeStruct((NCORES, np_, F), jnp.float32),
        mesh=_sc_mesh(),
        scratch_types=[
            pltpu.VMEM((ch, LANES), jnp.int32),
            pltpu.VMEM((ch, LANES), jnp.int32),
            pltpu.VMEM((LANES, F), jnp.float32),
            pltpu.MemorySpace.VMEM_SHARED((np_, F), jnp.float32),
            pltpu.SemaphoreType.DMA,
        ],
    )
    def sc_prop(g_hbm, row_hbm, col_hbm, zeros_hbm, out_hbm,
                row_v, col_v, rows_v, acc_sh, sem):
        c = lax.axis_index("c")
        s = lax.axis_index("s")
        slab = s * NCORES + c
        pltpu.sync_copy(row_hbm.at[slab], row_v)
        pltpu.sync_copy(col_hbm.at[slab], col_v)
        pltpu.sync_copy(zeros_hbm, acc_sh.at[pl.ds(s * rpt, rpt)])
        plsc.subcore_barrier()

        def body(j, carry):
            pltpu.async_copy(g_hbm.at[row_v.at[j]], rows_v, sem).wait()
            pltpu.sync_copy(rows_v, acc_sh.at[col_v.at[j]], add=True)
            return carry

        lax.fori_loop(0, ch, body, 0)
        plsc.subcore_barrier()
        pltpu.sync_copy(acc_sh.at[pl.ds(s * rpt, rpt)],
                        out_hbm.at[c, pl.ds(s * rpt, rpt)])

    return sc_prop


def _tc_feat(x_ref, g_ref, b_ref, w_ref, bias_ref, out_ref):
    h = _bn_rows(x_ref[...], g_ref[...], b_ref[...])
    out_ref[...] = jax.nn.relu(_mm(h, w_ref[...]) + bias_ref[...])


def _tc_first(n, np_):
    def body(h_ref, deg_ref, g_ref, b_ref, w_ref, bias_ref,
             gout_ref, dis_ref):
        deg = deg_ref[0, :n] + deg_ref[1, :n] - 1.0
        dis = jnp.where(deg > 0, jax.lax.rsqrt(deg), 0.0)
        t = _mm(_bn_rows(h_ref[...], g_ref[...], b_ref[...]), w_ref[...])
        g = dis * (t + bias_ref[...])
        pad = jnp.zeros((np_ - n, F), jnp.float32)
        gout_ref[...] = jnp.concatenate([g, pad], axis=0)
        dis_ref[...] = jnp.concatenate([dis, jnp.zeros((np_ - n, 1), jnp.float32)], axis=0)
    return body


def _tc_mid(n, np_):
    def body(acc_ref, gprev_ref, dis_ref, g_ref, b_ref, w_ref, bias_ref,
             gout_ref):
        dis = dis_ref[:n]
        ssum = gprev_ref[:n] + acc_ref[0, :n] + acc_ref[1, :n]
        h = jax.nn.relu(dis * ssum)
        t = _mm(_bn_rows(h, g_ref[...], b_ref[...]), w_ref[...])
        g = dis * (t + bias_ref[...])
        pad = jnp.zeros((np_ - n, F), jnp.float32)
        gout_ref[...] = jnp.concatenate([g, pad], axis=0)
    return body


def _tc_final(n, g_graphs):
    def body(acc_ref, gprev_ref, dis_ref, batch_ref,
             bnfc_g_ref, bnfc_b_ref, wfc_ref, bfc_ref,
             bnh_g_ref, bnh_b_ref, wcls_ref, bcls_ref, out_ref):
        dis = dis_ref[:n]
        ssum = gprev_ref[:n] + acc_ref[0, :n] + acc_ref[1, :n]
        h = jax.nn.relu(dis * ssum)
        gid = lax.broadcasted_iota(jnp.int32, (g_graphs, n), 0)
        onehot = (gid == batch_ref[...]).astype(jnp.float32)
        pooled = _mm(onehot, h)
        z = _bn_rows(pooled, bnfc_g_ref[...], bnfc_b_ref[...])
        z = jax.nn.relu(_mm(z, wfc_ref[...]) + bfc_ref[...])
        z = _bn_rows(z, bnh_g_ref[...], bnh_b_ref[...])
        logits = _mm(z, wcls_ref[...]) + bcls_ref[...]
        mx = jnp.max(logits, axis=-1, keepdims=True)
        lse = jnp.log(jnp.sum(jnp.exp(logits - mx), axis=-1, keepdims=True)) + mx
        out_ref[...] = logits - lse
    return body


def _call(body, out_shapes):
    return pl.pallas_call(body, out_shape=out_shapes)


def kernel(x, edge_index, batch, bnf_g, bnf_b, W_feat, b_feat, bns_g, bns_b,
           Ws, bs, bnfc_g, bnfc_b, W_fc, b_fc, bnh_g, bnh_b, W_cls, b_cls):
    n = x.shape[0]
    e = edge_index.shape[1]
    g_graphs = 64

    rpt = -(-(n + 1) // (NSUB * RPT_ALIGN)) * RPT_ALIGN
    np_ = rpt * NSUB
    ch = -(-e // (NCORES * NSUB * LANES))
    epad = ch * NCORES * NSUB * LANES

    row = edge_index[0]
    col = edge_index[1]
    padv = jnp.full((epad - e,), n, dtype=edge_index.dtype)
    rowp = jnp.concatenate([row, padv]).reshape(NCORES * NSUB, ch, LANES)
    colp = jnp.concatenate([col, padv]).reshape(NCORES * NSUB, ch, LANES)
    zeros2d = jnp.zeros((rpt, F), jnp.float32)
    ones1d = jnp.ones((rpt,), jnp.float32)
    batch_row = batch.reshape(1, n)

    r1 = lambda a: a.reshape(1, -1)

    sc_deg = _make_sc_deg(np_, ch, rpt)
    sc_prop = _make_sc_prop(np_, ch, rpt)

    deg = sc_deg(rowp, ones1d)
    deg3 = deg.reshape(NCORES, np_, 1)

    h0 = _call(_tc_feat, jax.ShapeDtypeStruct((n, F), jnp.float32))(
        x, r1(bnf_g), r1(bnf_b), W_feat, r1(b_feat))

    g0, dis = _call(
        _tc_first(n, np_),
        (jax.ShapeDtypeStruct((np_, F), jnp.float32),
         jax.ShapeDtypeStruct((np_, 1), jnp.float32)),
    )(h0, deg3, r1(bns_g[0]), r1(bns_b[0]), Ws[0], r1(bs[0]))

    g_cur = g0
    for i in range(1, Ws.shape[0]):
        acc = sc_prop(g_cur, rowp, colp, zeros2d)
        g_cur = _call(
            _tc_mid(n, np_), jax.ShapeDtypeStruct((np_, F), jnp.float32)
        )(acc, g_cur, dis, r1(bns_g[i]), r1(bns_b[i]), Ws[i], r1(bs[i]))

    acc = sc_prop(g_cur, rowp, colp, zeros2d)
    out = _call(
        _tc_final(n, g_graphs),
        jax.ShapeDtypeStruct((g_graphs, W_cls.shape[1]), jnp.float32),
    )(acc, g_cur, dis, batch_row,
      r1(bnfc_g), r1(bnfc_b), W_fc, r1(b_fc),
      r1(bnh_g), r1(bnh_b), W_cls, r1(b_cls))
    return out
